# baseline (speedup 1.0000x reference)
import functools

import jax
import jax.numpy as jnp
from jax import lax
from jax.experimental import pallas as pl
from jax.experimental.pallas import tpu as pltpu
from jax.experimental.pallas import tpu_sc as plsc


def _make_lookup(npairs, total):
    info = plsc.get_sparse_core_info()
    num_cores, num_subcores = info.num_cores, info.num_subcores
    num_workers = num_cores * num_subcores
    assert total % num_workers == 0
    n = total // num_workers
    ch = min(n, 256)
    assert n % ch == 0 and (n // ch) % 2 == 0

    mesh = plsc.VectorSubcoreMesh(core_axis_name="c", subcore_axis_name="s")

    @functools.partial(
        pl.kernel,
        mesh=mesh,
        compiler_params=pltpu.CompilerParams(use_tc_tiling_on_sc=True),
        out_type=jax.ShapeDtypeStruct((total, 128), jnp.float32),
        scratch_types=[
            pltpu.VMEM((n,), jnp.int32),
            pltpu.VMEM((ch, 128), jnp.float32),
            pltpu.VMEM((ch, 128), jnp.float32),
            pltpu.SemaphoreType.DMA,
            pltpu.SemaphoreType.DMA,
        ],
    )
    def lookup(idx_hbm, pairs_hbm, out_hbm, idx, rows0, rows1, sem0, sem1):
        wid = lax.axis_index("s") * num_cores + lax.axis_index("c")
        base = wid * n
        pltpu.sync_copy(idx_hbm.at[pl.ds(base, n)], idx)
        for c in range(0, n // ch, 2):
            cp0 = pltpu.async_copy(
                pairs_hbm.at[idx.at[pl.ds(c * ch, ch)]], rows0, sem0)
            cp1 = pltpu.async_copy(
                pairs_hbm.at[idx.at[pl.ds((c + 1) * ch, ch)]], rows1, sem1)
            cp0.wait()
            pltpu.sync_copy(rows0, out_hbm.at[pl.ds(base + c * ch, ch)])
            cp1.wait()
            pltpu.sync_copy(rows1, out_hbm.at[pl.ds(base + (c + 1) * ch, ch)])

    return lookup


def kernel(user, item, hash_embeds_weight):
    vocab, embed = hash_embeds_weight.shape
    (batch,) = user.shape
    npairs = (vocab - 1) // 2
    idx = jnp.stack([user, item], axis=1).reshape(-1)
    pidx = jnp.minimum(idx >> 1, npairs - 1)
    pairs = hash_embeds_weight[: 2 * npairs].reshape(npairs, 2 * embed)
    lookup = _make_lookup(npairs, 2 * batch)
    g = lookup(pidx, pairs)
    sel = jnp.where((idx & 1)[:, None] == 1, g[:, embed:], g[:, :embed])
    tail = hash_embeds_weight[vocab - 1]
    out = jnp.where((idx == vocab - 1)[:, None], tail[None, :], sel)
    return out.reshape(batch, 2 * embed)


# speedup vs baseline: 1.1595x; 1.1595x over previous
import functools

import jax
import jax.numpy as jnp
from jax import lax
from jax.experimental import pallas as pl
from jax.experimental.pallas import tpu as pltpu
from jax.experimental.pallas import tpu_sc as plsc


def _make_lookup(vocab, total):
    info = plsc.get_sparse_core_info()
    num_cores, num_subcores = info.num_cores, info.num_subcores
    num_workers = num_cores * num_subcores
    assert total % num_workers == 0
    n = total // num_workers
    ch = min(n, 256)
    assert n % ch == 0 and (n // ch) % 2 == 0

    mesh = plsc.VectorSubcoreMesh(core_axis_name="c", subcore_axis_name="s")

    @functools.partial(
        pl.kernel,
        mesh=mesh,
        compiler_params=pltpu.CompilerParams(use_tc_tiling_on_sc=True),
        out_type=jax.ShapeDtypeStruct((total, 128), jnp.float32),
        scratch_types=[
            pltpu.VMEM((n,), jnp.int32),
            pltpu.VMEM((ch, 128), jnp.float32),
            pltpu.VMEM((ch, 128), jnp.float32),
            pltpu.SemaphoreType.DMA,
            pltpu.SemaphoreType.DMA,
        ],
    )
    def lookup(idx_hbm, table_hbm, out_hbm, idx, rows0, rows1, sem0, sem1):
        wid = lax.axis_index("s") * num_cores + lax.axis_index("c")
        base = wid * n
        pltpu.sync_copy(idx_hbm.at[pl.ds(base, n)], idx)
        for c in range(0, n // ch, 2):
            cp0 = pltpu.async_copy(
                table_hbm.at[idx.at[pl.ds(c * ch, ch)]], rows0, sem0)
            cp1 = pltpu.async_copy(
                table_hbm.at[idx.at[pl.ds((c + 1) * ch, ch)]], rows1, sem1)
            cp0.wait()
            pltpu.sync_copy(rows0, out_hbm.at[pl.ds(base + c * ch, ch)])
            cp1.wait()
            pltpu.sync_copy(rows1, out_hbm.at[pl.ds(base + (c + 1) * ch, ch)])

    return lookup


def kernel(user, item, hash_embeds_weight):
    vocab, embed = hash_embeds_weight.shape
    (batch,) = user.shape
    idx = jnp.stack([user, item], axis=1).reshape(-1)
    tbl128 = jnp.pad(hash_embeds_weight, ((0, 0), (0, 128 - embed)))
    lookup = _make_lookup(vocab, 2 * batch)
    g = lookup(idx, tbl128)
    return g[:, :embed].reshape(batch, 2 * embed)
